# baseline (speedup 1.0000x reference)
import functools

import jax
import jax.numpy as jnp
from jax import lax
from jax.experimental import pallas as pl
from jax.experimental.pallas import tpu as pltpu
from jax.experimental.pallas import tpu_sc as plsc

_NC = 2
_NS = 16
_NW = _NC * _NS
_RSH = 3
_DSH = 4


def _gather_kernel(n_batch, table_rows, n_idx, d):
    per_t = n_batch // _NW
    half = n_idx // 2
    rpw = 128 // d
    srows = table_rows // rpw
    orows = half // rpw

    mesh = plsc.VectorSubcoreMesh(
        core_axis_name="c", subcore_axis_name="s",
        num_cores=_NC, num_subcores=_NS)

    @functools.partial(
        pl.kernel,
        out_type=jax.ShapeDtypeStruct((n_batch, 2 * orows, 128), jnp.float32),
        mesh=mesh,
        compiler_params=pltpu.CompilerParams(needs_layout_passes=False),
        scratch_types=[
            pltpu.VMEM((srows, 128), jnp.float32),
            pltpu.VMEM((n_idx,), jnp.int32),
            pltpu.VMEM((n_idx,), jnp.int32),
            pltpu.VMEM((n_idx,), jnp.int32),
            pltpu.VMEM((2, orows, 128), jnp.float32),
            pltpu.SemaphoreType.DMA,
            pltpu.SemaphoreType.DMA,
            pltpu.SemaphoreType.DMA,
            pltpu.SemaphoreType.DMA,
        ],
    )
    def k(table_hbm, idx_hbm, out_hbm, slab_v, idx_v, vrow_v, vcol_v, rows_v,
          isem, ssem, osem0, osem1):
        osem = (osem0, osem1)
        wid = lax.axis_index("s") * _NC + lax.axis_index("c")
        b0 = wid * per_t

        pltpu.async_copy(idx_hbm, idx_v, isem).wait()

        def dec(q, _):
            sl = pl.ds(q * 16, 16)
            v = idx_v[sl]
            vrow_v[sl] = lax.shift_right_logical(v, _RSH)
            vcol_v[sl] = lax.shift_left(jnp.bitwise_and(v, rpw - 1), _DSH)
            return 0
        lax.fori_loop(0, n_idx // 16, dec, 0)

        def slab_src(b):
            return table_hbm.at[b0 + b]

        def fire_slab(b):
            pltpu.async_copy(slab_src(b), slab_v, ssem)

        def wait_slab(b):
            pltpu.make_async_copy(slab_src(b), slab_v, ssem).wait()

        lanes = lax.iota(jnp.int32, 16)
        oconst = lax.shift_right_logical(lanes, _RSH)
        ocol0 = lax.shift_left(jnp.bitwise_and(lanes, rpw - 1), _DSH)
        rows2d = [rows_v.at[0], rows_v.at[1]]

        def gather_half(h):
            def grp(q, _):
                sl = pl.ds((h * half // 16 + q) * 16, 16)
                r = vrow_v[sl]
                c = vcol_v[sl]
                orow = oconst + q * (16 // rpw)
                for kcol in range(d):
                    vals = plsc.load_gather(slab_v, [r, c + kcol])
                    plsc.store_scatter(rows2d[h], [orow, ocol0 + kcol], vals)
                return 0
            lax.fori_loop(0, half // 16, grp, 0)

        def scat_dst(h, b):
            return out_hbm.at[b0 + b, pl.ds(h * orows, orows), :]

        def fire_scat(h, b):
            pltpu.async_copy(rows_v.at[h], scat_dst(h, b), osem[h])

        def wait_scat(h, b):
            pltpu.make_async_copy(rows_v.at[h], scat_dst(h, b), osem[h]).wait()

        fire_slab(0)
        wait_slab(0)
        for h in range(2):
            gather_half(h)
            fire_scat(h, 0)
        fire_slab(1)

        def body(t, _):
            b = 1 + t
            wait_slab(b)
            for h in range(2):
                wait_scat(h, b - 1)
                gather_half(h)
                fire_scat(h, b)
            fire_slab(b + 1)
            return 0
        lax.fori_loop(0, per_t - 2, body, 0)

        b_last = per_t - 1
        wait_slab(b_last)
        for h in range(2):
            wait_scat(h, b_last - 1)
            gather_half(h)
            fire_scat(h, b_last)
        for h in range(2):
            wait_scat(h, b_last)

    return k


def kernel(params, idx):
    n_batch, table_rows, d = params.shape
    i = idx[0, :, 0].astype(jnp.int32)
    n_idx = i.shape[0]
    table128 = params.reshape(n_batch, table_rows * d // 128, 128)
    out = _gather_kernel(n_batch, table_rows, n_idx, d)(table128, i)
    return out.reshape(n_batch, n_idx, d)


# speedup vs baseline: 1.2963x; 1.2963x over previous
import functools

import jax
import jax.numpy as jnp
from jax import lax
from jax.experimental import pallas as pl
from jax.experimental.pallas import tpu as pltpu
from jax.experimental.pallas import tpu_sc as plsc

_NC = 2
_NS = 16
_NW = _NC * _NS


def _gather_kernel(n_batch, table_rows, n_idx, d):
    per_t = n_batch // _NW
    half = n_idx // 2
    slab_w = table_rows * d
    half_w = half * d

    mesh = plsc.VectorSubcoreMesh(
        core_axis_name="c", subcore_axis_name="s",
        num_cores=_NC, num_subcores=_NS)

    @functools.partial(
        pl.kernel,
        out_type=jax.ShapeDtypeStruct((n_batch, n_idx * d), jnp.float32),
        mesh=mesh,
        compiler_params=pltpu.CompilerParams(needs_layout_passes=False),
        scratch_types=[
            pltpu.VMEM((slab_w,), jnp.float32),
            pltpu.VMEM((n_idx,), jnp.int32),
            pltpu.VMEM((n_idx,), jnp.int32),
            pltpu.VMEM((half_w,), jnp.float32),
            pltpu.VMEM((half_w,), jnp.float32),
            pltpu.SemaphoreType.DMA,
            pltpu.SemaphoreType.DMA,
            pltpu.SemaphoreType.DMA,
            pltpu.SemaphoreType.DMA,
        ],
    )
    def k(table_hbm, idx_hbm, out_hbm, slab_v, idx_v, off_v, rows0_v,
          rows1_v, isem, ssem, osem0, osem1):
        rows = (rows0_v, rows1_v)
        osem = (osem0, osem1)
        wid = lax.axis_index("s") * _NC + lax.axis_index("c")
        b0 = wid * per_t

        pltpu.async_copy(idx_hbm, idx_v, isem).wait()

        def dec(q, _):
            sl = pl.ds(q * 16, 16)
            off_v[sl] = lax.shift_left(idx_v[sl], d.bit_length() - 1)
            return 0
        lax.fori_loop(0, n_idx // 16, dec, 0)

        lanes = lax.iota(jnp.int32, 16)
        lanes_d = lanes * d
        diags = [jnp.bitwise_and(lanes + m, d - 1) for m in range(d)]

        def fire_slab(b):
            pltpu.async_copy(table_hbm.at[b0 + b], slab_v, ssem)

        def wait_slab(b):
            pltpu.make_async_copy(table_hbm.at[b0 + b], slab_v, ssem).wait()

        def gather_half(h):
            def grp(q, _):
                v16 = off_v[pl.ds((h * half // 16 + q) * 16, 16)]
                j16 = lanes_d + q * (16 * d)
                for m in range(d):
                    vals = plsc.load_gather(slab_v, [v16 + diags[m]])
                    plsc.store_scatter(rows[h], [j16 + diags[m]], vals)
                return 0
            lax.fori_loop(0, half // 16, grp, 0)

        def fire_scat(h, b):
            pltpu.async_copy(
                rows[h], out_hbm.at[b0 + b, pl.ds(h * half_w, half_w)],
                osem[h])

        def wait_scat(h, b):
            pltpu.make_async_copy(
                rows[h], out_hbm.at[b0 + b, pl.ds(h * half_w, half_w)],
                osem[h]).wait()

        fire_slab(0)
        wait_slab(0)
        for h in range(2):
            gather_half(h)
            fire_scat(h, 0)
        fire_slab(1)

        def body(t, _):
            b = 1 + t
            wait_slab(b)
            for h in range(2):
                wait_scat(h, b - 1)
                gather_half(h)
                fire_scat(h, b)
            fire_slab(b + 1)
            return 0
        lax.fori_loop(0, per_t - 2, body, 0)

        b_last = per_t - 1
        wait_slab(b_last)
        for h in range(2):
            wait_scat(h, b_last - 1)
            gather_half(h)
            fire_scat(h, b_last)
        for h in range(2):
            wait_scat(h, b_last)

    return k


def kernel(params, idx):
    n_batch, table_rows, d = params.shape
    i = idx[0, :, 0].astype(jnp.int32)
    n_idx = i.shape[0]
    table_flat = params.reshape(n_batch, table_rows * d)
    out = _gather_kernel(n_batch, table_rows, n_idx, d)(table_flat, i)
    return out.reshape(n_batch, n_idx, d)


# speedup vs baseline: 1.5626x; 1.2055x over previous
import functools

import jax
import jax.numpy as jnp
from jax import lax
from jax.experimental import pallas as pl
from jax.experimental.pallas import tpu as pltpu
from jax.experimental.pallas import tpu_sc as plsc

_NC = 2
_NS = 16
_NW = _NC * _NS


def _gather_kernel(n_batch, table_rows, n_idx, d):
    per_t = n_batch // _NW
    half = n_idx // 2
    slab_w = table_rows * d
    half_w = half * d

    mesh = plsc.VectorSubcoreMesh(
        core_axis_name="c", subcore_axis_name="s",
        num_cores=_NC, num_subcores=_NS)

    @functools.partial(
        pl.kernel,
        out_type=jax.ShapeDtypeStruct((n_batch, n_idx * d), jnp.float32),
        mesh=mesh,
        compiler_params=pltpu.CompilerParams(needs_layout_passes=False),
        scratch_types=[
            pltpu.VMEM((slab_w,), jnp.float32),
            pltpu.VMEM((n_idx,), jnp.int32),
            pltpu.VMEM((n_idx,), jnp.int32),
            pltpu.VMEM((half_w,), jnp.float32),
            pltpu.VMEM((half_w,), jnp.float32),
            pltpu.SemaphoreType.DMA,
            pltpu.SemaphoreType.DMA,
            pltpu.SemaphoreType.DMA,
            pltpu.SemaphoreType.DMA,
        ],
    )
    def k(table_hbm, idx_hbm, out_hbm, slab_v, idx_v, off_v, rows0_v,
          rows1_v, isem, ssem, osem0, osem1):
        rows = (rows0_v, rows1_v)
        osem = (osem0, osem1)
        wid = lax.axis_index("s") * _NC + lax.axis_index("c")
        b0 = wid * per_t

        pltpu.async_copy(idx_hbm, idx_v, isem).wait()

        def dec(q, _):
            sl = pl.ds(q * 16, 16)
            off_v[sl] = lax.shift_left(idx_v[sl], d.bit_length() - 1)
            return 0
        lax.fori_loop(0, n_idx // 16, dec, 0)

        lanes = lax.iota(jnp.int32, 16)
        lanes_d = lanes * d
        diags = [jnp.bitwise_and(lanes + m, d - 1) for m in range(d)]

        def fire_slab(b):
            pltpu.async_copy(table_hbm.at[b0 + b], slab_v, ssem)

        def wait_slab(b):
            pltpu.make_async_copy(table_hbm.at[b0 + b], slab_v, ssem).wait()

        def gather_half(h):
            def grp(q, _):
                v16 = off_v[pl.ds((h * half // 16 + q) * 16, 16)]
                j16 = lanes_d + q * (16 * d)
                vals = [plsc.load_gather(slab_v, [v16 + diags[m]])
                        for m in range(d)]
                for m in range(d):
                    plsc.store_scatter(rows[h], [j16 + diags[m]], vals[m])
                return 0
            lax.fori_loop(0, half // 16, grp, 0)

        def fire_scat(h, b):
            pltpu.async_copy(
                rows[h], out_hbm.at[b0 + b, pl.ds(h * half_w, half_w)],
                osem[h])

        def wait_scat(h, b):
            pltpu.make_async_copy(
                rows[h], out_hbm.at[b0 + b, pl.ds(h * half_w, half_w)],
                osem[h]).wait()

        fire_slab(0)
        wait_slab(0)
        for h in range(2):
            gather_half(h)
            fire_scat(h, 0)
        fire_slab(1)

        def body(t, _):
            b = 1 + t
            wait_slab(b)
            for h in range(2):
                wait_scat(h, b - 1)
                gather_half(h)
                fire_scat(h, b)
            fire_slab(b + 1)
            return 0
        lax.fori_loop(0, per_t - 2, body, 0)

        b_last = per_t - 1
        wait_slab(b_last)
        for h in range(2):
            wait_scat(h, b_last - 1)
            gather_half(h)
            fire_scat(h, b_last)
        for h in range(2):
            wait_scat(h, b_last)

    return k


def kernel(params, idx):
    n_batch, table_rows, d = params.shape
    i = idx[0, :, 0].astype(jnp.int32)
    n_idx = i.shape[0]
    table_flat = params.reshape(n_batch, table_rows * d)
    out = _gather_kernel(n_batch, table_rows, n_idx, d)(table_flat, i)
    return out.reshape(n_batch, n_idx, d)
